# Initial kernel scaffold; baseline (speedup 1.0000x reference)
import functools

import jax
import jax.numpy as jnp
from jax import lax
from jax.experimental import pallas as pl
from jax.experimental.pallas import tpu as pltpu
from jax.experimental.pallas import tpu_sc as plsc

BATCH = 16384
HIST = 200
VPAD = 24
NUM_WORKERS = 32
ROWS_PER_W = BATCH // NUM_WORKERS
GROUPS_PER_W = ROWS_PER_W // 16
L_UNROLL = 8
L_CHUNKS = HIST // L_UNROLL


def _hist_body(x_hbm, counts_hbm, x_v, counts_v):
    nc = 2
    wid = lax.axis_index("s") * nc + lax.axis_index("c")

    pltpu.sync_copy(x_hbm.at[pl.ds(wid * (ROWS_PER_W * HIST), ROWS_PER_W * HIST)], x_v)

    zeros = jnp.zeros((16,), jnp.float32)

    def zero_body(i, carry):
        counts_v[pl.ds(i * 16, 16)] = zeros
        return carry

    lax.fori_loop(0, ROWS_PER_W * VPAD // 16, zero_body, 0)

    iota16 = lax.iota(jnp.int32, 16)
    ones = jnp.ones((16,), jnp.float32)

    def group_body(jg, carry):
        rows = iota16 + jg * 16
        rows_x = rows * HIST
        rows_c = rows * VPAD

        def l_body(c, carry2):
            l0 = c * L_UNROLL
            for u in range(L_UNROLL):
                ids = plsc.load_gather(x_v, [rows_x + (l0 + u)])
                plsc.addupdate_scatter(counts_v, [rows_c + ids], ones)
            return carry2

        lax.fori_loop(0, L_CHUNKS, l_body, 0)
        return carry

    lax.fori_loop(0, GROUPS_PER_W, group_body, 0)

    pltpu.sync_copy(counts_v, counts_hbm.at[pl.ds(wid * (ROWS_PER_W * VPAD), ROWS_PER_W * VPAD)])


@functools.partial(
    pl.kernel,
    mesh=plsc.VectorSubcoreMesh(core_axis_name="c", subcore_axis_name="s"),
    out_type=jax.ShapeDtypeStruct((BATCH * VPAD,), jnp.float32),
    scratch_types=[
        pltpu.VMEM((ROWS_PER_W * HIST,), jnp.int32),
        pltpu.VMEM((ROWS_PER_W * VPAD,), jnp.float32),
    ],
)
def _hist(x_hbm, counts_hbm, x_v, counts_v):
    _hist_body(x_hbm, counts_hbm, x_v, counts_v)


def _matmul_body(c_ref, t_ref, o_ref):
    o_ref[:, :] = jnp.tanh(
        jnp.dot(c_ref[:, :], t_ref[:, :], preferred_element_type=jnp.float32)
    )


def _matmul_tanh(counts, table24):
    blk = 2048
    return pl.pallas_call(
        _matmul_body,
        grid=(BATCH // blk,),
        in_specs=[
            pl.BlockSpec((blk, VPAD), lambda i: (i, 0)),
            pl.BlockSpec((VPAD, 128), lambda i: (0, 0)),
        ],
        out_specs=pl.BlockSpec((blk, 128), lambda i: (i, 0)),
        out_shape=jax.ShapeDtypeStruct((BATCH, 128), jnp.float32),
    )(counts, table24)


def kernel(x, table):
    counts = _hist(x.reshape(-1)).reshape(BATCH, VPAD)
    table24 = jnp.concatenate([table, jnp.zeros((1, 128), table.dtype)], axis=0)
    return _matmul_tanh(counts, table24)


# speedup vs baseline: 94.5013x; 94.5013x over previous
import functools

import jax
import jax.numpy as jnp
from jax import lax
from jax.experimental import pallas as pl
from jax.experimental.pallas import tpu as pltpu
from jax.experimental.pallas import tpu_sc as plsc

BATCH = 16384
HIST = 200
VPAD = 24
NUM_WORKERS = 32
ROWS_PER_W = BATCH // NUM_WORKERS
GROUPS_PER_W = ROWS_PER_W // 16
L_UNROLL = 8
L_CHUNKS = HIST // L_UNROLL


def _hist_body(x_hbm, counts_hbm, x_v, counts_v):
    nc = 2
    wid = lax.axis_index("s") * nc + lax.axis_index("c")

    pltpu.sync_copy(x_hbm.at[pl.ds(wid * ROWS_PER_W, ROWS_PER_W), :], x_v)

    zeros = jnp.zeros((16,), jnp.float32)

    def zero_body(i, carry):
        counts_v[i, pl.ds(0, 16)] = zeros
        counts_v[i, pl.ds(8, 16)] = zeros
        return carry

    lax.fori_loop(0, ROWS_PER_W, zero_body, 0)

    iota16 = lax.iota(jnp.int32, 16)
    ones = jnp.ones((16,), jnp.float32)

    def group_body(jg, carry):
        rows = iota16 + jg * 16

        def l_body(c, carry2):
            l0 = c * L_UNROLL
            for u in range(L_UNROLL):
                lcol = jnp.full((16,), l0 + u, jnp.int32)
                ids = plsc.load_gather(x_v, [rows, lcol])
                plsc.addupdate_scatter(counts_v, [rows, ids], ones)
            return carry2

        lax.fori_loop(0, L_CHUNKS, l_body, 0)
        return carry

    lax.fori_loop(0, GROUPS_PER_W, group_body, 0)

    pltpu.sync_copy(counts_v, counts_hbm.at[pl.ds(wid * ROWS_PER_W, ROWS_PER_W), :])


@functools.cache
def _make_hist():
    return pl.kernel(
        _hist_body,
        mesh=plsc.VectorSubcoreMesh(core_axis_name="c", subcore_axis_name="s"),
        out_type=jax.ShapeDtypeStruct((BATCH, VPAD), jnp.float32),
        scratch_types=[
            pltpu.VMEM((ROWS_PER_W, HIST), jnp.int32),
            pltpu.VMEM((ROWS_PER_W, VPAD), jnp.float32),
        ],
        compiler_params=pltpu.CompilerParams(
            use_tc_tiling_on_sc=False,
            needs_layout_passes=False,
        ),
    )


def _matmul_body(c_ref, t_ref, o_ref):
    o_ref[:, :] = jnp.tanh(
        jnp.dot(
            c_ref[:, :],
            t_ref[:, :],
            preferred_element_type=jnp.float32,
            precision=lax.Precision.HIGHEST,
        )
    )


def _matmul_tanh(counts, table24):
    blk = 2048
    return pl.pallas_call(
        _matmul_body,
        grid=(BATCH // blk,),
        in_specs=[
            pl.BlockSpec((blk, VPAD), lambda i: (i, 0)),
            pl.BlockSpec((VPAD, 128), lambda i: (0, 0)),
        ],
        out_specs=pl.BlockSpec((blk, 128), lambda i: (i, 0)),
        out_shape=jax.ShapeDtypeStruct((BATCH, 128), jnp.float32),
    )(counts, table24)


def kernel(x, table):
    counts = _make_hist()(x)
    table24 = jnp.concatenate([table, jnp.zeros((1, 128), table.dtype)], axis=0)
    return _matmul_tanh(counts, table24)


# speedup vs baseline: 94.7041x; 1.0021x over previous
import functools

import jax
import jax.numpy as jnp
from jax import lax
from jax.experimental import pallas as pl
from jax.experimental.pallas import tpu as pltpu
from jax.experimental.pallas import tpu_sc as plsc

BATCH = 16384
HIST = 200
VPAD = 24
NUM_WORKERS = 32
ROWS_PER_W = BATCH // NUM_WORKERS
GROUPS_PER_W = ROWS_PER_W // 16
L_UNROLL = 8
L_CHUNKS = HIST // L_UNROLL


NBUF = 4
HALF = ROWS_PER_W // 2


def _hist_body(x_hbm, counts_hbm, x_v, *bufs):
    nc = 2
    wid = lax.axis_index("s") * nc + lax.axis_index("c")

    zeros = jnp.zeros((16,), jnp.float32)
    iota16 = lax.iota(jnp.int32, 16)
    ones = jnp.ones((16,), jnp.float32)

    def zero_body(i, carry):
        for b in bufs:
            b[pl.ds(i * 16, 16)] = zeros
        return carry

    lax.fori_loop(0, ROWS_PER_W * VPAD // 16, zero_body, 0)

    for half in range(2):
        pltpu.sync_copy(
            x_hbm.at[pl.ds((wid * ROWS_PER_W + half * HALF) * HIST, HALF * HIST)], x_v
        )

        def group_body(jg, carry):
            rows_x = (iota16 + jg * 16) * HIST
            rows_c = (iota16 + jg * 16 + half * HALF) * VPAD

            def l_body(c, carry2):
                l0 = c * L_UNROLL
                for u in range(L_UNROLL):
                    ids = plsc.load_gather(x_v, [rows_x + (l0 + u)])
                    plsc.addupdate_scatter(bufs[u % NBUF], [rows_c + ids], ones)
                return carry2

            lax.fori_loop(0, L_CHUNKS, l_body, 0)
            return carry

        lax.fori_loop(0, HALF // 16, group_body, 0)

    def merge_body(i, carry):
        acc = bufs[0][pl.ds(i * 16, 16)]
        for b in bufs[1:]:
            acc = acc + b[pl.ds(i * 16, 16)]
        bufs[0][pl.ds(i * 16, 16)] = acc
        return carry

    lax.fori_loop(0, ROWS_PER_W * VPAD // 16, merge_body, 0)

    pltpu.sync_copy(
        bufs[0], counts_hbm.at[pl.ds(wid * (ROWS_PER_W * VPAD), ROWS_PER_W * VPAD)]
    )


@functools.cache
def _make_hist():
    return pl.kernel(
        _hist_body,
        mesh=plsc.VectorSubcoreMesh(core_axis_name="c", subcore_axis_name="s"),
        out_type=jax.ShapeDtypeStruct((BATCH * VPAD,), jnp.float32),
        scratch_types=[
            pltpu.VMEM((HALF * HIST,), jnp.int32),
        ]
        + [pltpu.VMEM((ROWS_PER_W * VPAD,), jnp.float32) for _ in range(NBUF)],
        compiler_params=pltpu.CompilerParams(
            use_tc_tiling_on_sc=False,
            needs_layout_passes=False,
        ),
    )


def _matmul_body(c_ref, t_ref, o_ref):
    o_ref[:, :] = jnp.tanh(
        jnp.dot(
            c_ref[:, :],
            t_ref[:, :],
            preferred_element_type=jnp.float32,
            precision=lax.Precision.HIGHEST,
        )
    )


def _matmul_tanh(counts, table24):
    blk = 2048
    return pl.pallas_call(
        _matmul_body,
        grid=(BATCH // blk,),
        in_specs=[
            pl.BlockSpec((blk, VPAD), lambda i: (i, 0)),
            pl.BlockSpec((VPAD, 128), lambda i: (0, 0)),
        ],
        out_specs=pl.BlockSpec((blk, 128), lambda i: (i, 0)),
        out_shape=jax.ShapeDtypeStruct((BATCH, 128), jnp.float32),
    )(counts, table24)


def kernel(x, table):
    counts = _make_hist()(x.reshape(-1)).reshape(BATCH, VPAD)
    table24 = jnp.concatenate([table, jnp.zeros((1, 128), table.dtype)], axis=0)
    return _matmul_tanh(counts, table24)


# speedup vs baseline: 141.6521x; 1.4957x over previous
import functools

import jax
import jax.numpy as jnp
from jax import lax
from jax.experimental import pallas as pl
from jax.experimental.pallas import tpu as pltpu
from jax.experimental.pallas import tpu_sc as plsc

BATCH = 16384
HIST = 200
VPAD = 24
NUM_WORKERS = 32
ROWS_PER_W = BATCH // NUM_WORKERS
HALF = ROWS_PER_W // 2
NBUF = 4


def _hist_body(x_hbm, counts_hbm, x_v, *bufs):
    nc = 2
    wid = lax.axis_index("s") * nc + lax.axis_index("c")

    zeros = jnp.zeros((16,), jnp.float32)
    iota16 = lax.iota(jnp.int32, 16)
    ones = jnp.ones((16,), jnp.float32)

    @plsc.parallel_loop(0, ROWS_PER_W)
    def _zero(i):
        for b in bufs:
            b[i, pl.ds(0, 16)] = zeros
            b[i, pl.ds(8, 16)] = zeros

    for half in range(2):
        pltpu.sync_copy(x_hbm.at[pl.ds(wid * ROWS_PER_W + half * HALF, HALF), :], x_v)

        @plsc.parallel_loop(0, HALF // 16)
        def _groups(jg):
            rows = iota16 + jg * 16
            rows_c = rows + half * HALF

            @plsc.parallel_loop(0, HIST, step=NBUF, unroll=2)
            def _hist_l(l0):
                for u in range(NBUF):
                    ids = plsc.load_gather(x_v, [rows, jnp.full((16,), l0 + u, jnp.int32)])
                    plsc.addupdate_scatter(bufs[u], [rows_c, ids], ones)

    @plsc.parallel_loop(0, ROWS_PER_W)
    def _merge(i):
        lo = bufs[0][i, pl.ds(0, 16)]
        hi = bufs[0][i, pl.ds(8, 16)]
        for b in bufs[1:]:
            lo = lo + b[i, pl.ds(0, 16)]
            hi = hi + b[i, pl.ds(8, 16)]
        bufs[0][i, pl.ds(0, 16)] = lo
        bufs[0][i, pl.ds(8, 16)] = hi

    pltpu.sync_copy(bufs[0], counts_hbm.at[pl.ds(wid * ROWS_PER_W, ROWS_PER_W), :])


@functools.cache
def _make_hist():
    return pl.kernel(
        _hist_body,
        mesh=plsc.VectorSubcoreMesh(core_axis_name="c", subcore_axis_name="s"),
        out_type=jax.ShapeDtypeStruct((BATCH, VPAD), jnp.float32),
        scratch_types=[
            pltpu.VMEM((HALF, HIST), jnp.int32),
        ]
        + [pltpu.VMEM((ROWS_PER_W, VPAD), jnp.float32) for _ in range(NBUF)],
        compiler_params=pltpu.CompilerParams(
            use_tc_tiling_on_sc=False,
            needs_layout_passes=False,
        ),
    )


def _matmul_body(c_ref, t_ref, o_ref):
    o_ref[:, :] = jnp.tanh(
        jnp.dot(
            c_ref[:, :],
            t_ref[:, :],
            preferred_element_type=jnp.float32,
            precision=lax.Precision.HIGHEST,
        )
    )


def _matmul_tanh(counts, table24):
    blk = 2048
    return pl.pallas_call(
        _matmul_body,
        grid=(BATCH // blk,),
        in_specs=[
            pl.BlockSpec((blk, VPAD), lambda i: (i, 0)),
            pl.BlockSpec((VPAD, 128), lambda i: (0, 0)),
        ],
        out_specs=pl.BlockSpec((blk, 128), lambda i: (i, 0)),
        out_shape=jax.ShapeDtypeStruct((BATCH, 128), jnp.float32),
    )(counts, table24)


def kernel(x, table):
    counts = _make_hist()(x)
    table24 = jnp.concatenate([table, jnp.zeros((1, 128), table.dtype)], axis=0)
    return _matmul_tanh(counts, table24)
